# baseline (speedup 1.0000x reference)
import functools

import jax
import jax.numpy as jnp
from jax import lax
from jax.experimental import pallas as pl
from jax.experimental.pallas import tpu as pltpu
from jax.experimental.pallas import tpu_sc as plsc

_TI = 1408
_NI = 2
_T = 16
_E = 16
_H = 1024


def _moe_body(x_ref, rw_ref, g_ref, u_ref, d_ref, eo_ref, logits_ref):
    e = pl.program_id(0)
    i = pl.program_id(1)

    @pl.when(jnp.logical_and(e == 0, i == 0))
    def _router():
        logits_ref[...] = jax.lax.dot_general(
            x_ref[...], rw_ref[...], (((1,), (1,)), ((), ())),
            preferred_element_type=jnp.float32)

    x = x_ref[...]
    g = jax.lax.dot_general(x, g_ref[0], (((1,), (1,)), ((), ())),
                            preferred_element_type=jnp.float32)
    u = jax.lax.dot_general(x, u_ref[0], (((1,), (1,)), ((), ())),
                            preferred_element_type=jnp.float32)
    h = g * jax.lax.logistic(g) * u
    contrib = jax.lax.dot_general(h, d_ref[0], (((1,), (1,)), ((), ())),
                                  preferred_element_type=jnp.float32)

    @pl.when(i == 0)
    def _init():
        eo_ref[0] = contrib

    @pl.when(i != 0)
    def _acc():
        eo_ref[0] += contrib


def _tc_experts(x, router_w, gate_w, up_w, down_w):
    t, h = x.shape
    e, i_dim, _ = gate_w.shape
    return pl.pallas_call(
        _moe_body,
        grid=(e, _NI),
        in_specs=[
            pl.BlockSpec((t, h), lambda e_, i_: (0, 0)),
            pl.BlockSpec((e, h), lambda e_, i_: (0, 0)),
            pl.BlockSpec((1, _TI, h), lambda e_, i_: (e_, i_, 0)),
            pl.BlockSpec((1, _TI, h), lambda e_, i_: (e_, i_, 0)),
            pl.BlockSpec((1, h, _TI), lambda e_, i_: (e_, 0, i_)),
        ],
        out_specs=[
            pl.BlockSpec((1, t, h), lambda e_, i_: (e_, 0, 0)),
            pl.BlockSpec((t, e), lambda e_, i_: (0, 0)),
        ],
        out_shape=[
            jax.ShapeDtypeStruct((e, t, h), x.dtype),
            jax.ShapeDtypeStruct((t, e), jnp.float32),
        ],
    )(x, router_w, gate_w, up_w, down_w)


_sc_cache = {}


def _get_sc_route_combine():
    if "k" in _sc_cache:
        return _sc_cache["k"]
    mesh = plsc.VectorSubcoreMesh(core_axis_name="c", subcore_axis_name="s")

    @functools.partial(
        pl.kernel,
        mesh=mesh,
        out_type=jax.ShapeDtypeStruct((_T, _H), jnp.float32),
        scratch_types=[
            pltpu.VMEM((_E,), jnp.float32),
            pltpu.VMEM((_H,), jnp.float32),
            pltpu.VMEM((_H,), jnp.float32),
            pltpu.VMEM((_H,), jnp.float32),
        ],
    )
    def _sc_route_combine(logits_hbm, eo_hbm, out_hbm, lrow, r1, r2, orow):
        cid = lax.axis_index("c")
        sid = lax.axis_index("s")
        tok = sid * 2 + cid

        @pl.when(tok < _T)
        def _():
            pltpu.sync_copy(logits_hbm.at[tok], lrow)
            vec = lrow[...]
            m1 = jnp.float32(-3.4e38)
            m2 = jnp.float32(-3.4e38)
            i1 = jnp.int32(0)
            i2 = jnp.int32(0)
            for k in range(_E):
                v = vec[k]
                is1 = v > m1
                is2 = jnp.logical_and(jnp.logical_not(is1), v > m2)
                m2 = jnp.where(is1, m1, jnp.where(is2, v, m2))
                i2 = jnp.where(is1, i1, jnp.where(is2, jnp.int32(k), i2))
                m1 = jnp.where(is1, v, m1)
                i1 = jnp.where(is1, jnp.int32(k), i1)
            pltpu.sync_copy(eo_hbm.at[i1 * _T + tok], r1)
            pltpu.sync_copy(eo_hbm.at[i2 * _T + tok], r2)
            for j in range(_H // 16):
                sl = pl.ds(16 * j, 16)
                orow[sl] = 0.25 * (r1[sl] + r2[sl])
            pltpu.sync_copy(orow, out_hbm.at[tok])

    _sc_cache["k"] = _sc_route_combine
    return _sc_route_combine


def kernel(x, router_w, gate_w, up_w, down_w):
    eo, logits = _tc_experts(x, router_w, gate_w, up_w, down_w)
    return _get_sc_route_combine()(logits, eo.reshape(_E * _T, _H))
